# baseline (device time: 425830 ns/iter reference)
import os as _os

import jax
import jax.numpy as jnp
from jax import lax
from jax.experimental import pallas as pl
from jax.experimental.pallas import tpu as pltpu

MESH = pl.DeviceIdType.MESH

K = int(_os.environ.get("AR_K", "32"))
LAG = 2
AHEAD = 8


def kernel(x):
    M, N = x.shape
    HALF = M // 2
    C = HALF // K

    def body(x_hbm, out_hbm, comm_hbm, ssem1, rsem1, ssem2, rsem2,
             vx, vc, vo, lx, lc, st):
        my_x = lax.axis_index("x")
        my_y = lax.axis_index("y")
        y_nbr = (my_x, 1 - my_y)
        x_nbr = (1 - my_x, my_y)

        barrier = pltpu.get_barrier_semaphore()
        for nbr in (y_nbr, x_nbr):
            pl.semaphore_signal(barrier, inc=1, device_id=nbr,
                                device_id_type=MESH)
        pl.semaphore_wait(barrier, 2)

        def rows_mine(k):
            return pl.ds(my_x * HALF + k * C, C)

        def rows_other(k):
            return pl.ds((1 - my_x) * HALF + k * C, C)

        rdma1 = []

        def send1(k):
            r = pltpu.make_async_remote_copy(
                src_ref=x_hbm.at[rows_mine(k), :],
                dst_ref=comm_hbm.at[rows_mine(k), :],
                send_sem=ssem1.at[k],
                recv_sem=rsem1.at[k],
                device_id=y_nbr,
                device_id_type=MESH,
            )
            r.start()
            rdma1.append(r)

        for k in range(min(AHEAD, K)):
            send1(k)

        rdma2 = []

        def forward(k):
            r = pltpu.make_async_remote_copy(
                src_ref=comm_hbm.at[rows_mine(k), :],
                dst_ref=comm_hbm.at[rows_mine(k), :],
                send_sem=ssem2.at[k],
                recv_sem=rsem2.at[k],
                device_id=x_nbr,
                device_id_type=MESH,
            )
            r.start()
            rdma2.append(r)

        tasks = []
        for k in range(K):
            tasks.append(("p1", k))
            if k >= LAG:
                tasks.append(("p2", k - LAG))
        for k in range(K - LAG, K):
            tasks.append(("p2", k))

        def task_rows(t):
            phase, k = tasks[t]
            return rows_mine(k) if phase == "p1" else rows_other(k)

        def start_xload(t):
            s = t % 2
            pltpu.make_async_copy(x_hbm.at[task_rows(t), :], vx.at[s],
                                  lx.at[s]).start()

        start_xload(0)
        for t in range(len(tasks)):
            s = t % 2
            phase, k = tasks[t]
            if phase == "p1":
                if k + AHEAD < K:
                    send1(k + AHEAD)
                rdma1[k].wait_recv()
                forward(k)
            else:
                rdma2[k].wait_recv()
            ld_c = pltpu.make_async_copy(comm_hbm.at[task_rows(t), :],
                                         vc.at[s], lc.at[s])
            ld_c.start()
            if t + 1 < len(tasks):
                start_xload(t + 1)
            pltpu.make_async_copy(x_hbm.at[task_rows(t), :], vx.at[s],
                                  lx.at[s]).wait()
            ld_c.wait()
            if t >= 2:
                pltpu.make_async_copy(vo.at[s], out_hbm.at[task_rows(t - 2), :],
                                      st.at[s]).wait()
            vo[s] = vx[s] + vc[s]
            pltpu.make_async_copy(vo.at[s], out_hbm.at[task_rows(t), :],
                                  st.at[s]).start()

        n_t = len(tasks)
        for t in (n_t - 2, n_t - 1):
            s = t % 2
            pltpu.make_async_copy(vo.at[s], out_hbm.at[task_rows(t), :],
                                  st.at[s]).wait()
        for k in range(K):
            rdma1[k].wait_send()
            rdma2[k].wait_send()

    out, _comm = pl.pallas_call(
        body,
        out_shape=[
            jax.ShapeDtypeStruct((M, N), jnp.float32),
            jax.ShapeDtypeStruct((M, N), jnp.float32),
        ],
        in_specs=[pl.BlockSpec(memory_space=pltpu.HBM)],
        out_specs=[
            pl.BlockSpec(memory_space=pltpu.HBM),
            pl.BlockSpec(memory_space=pltpu.HBM),
        ],
        scratch_shapes=[
            pltpu.SemaphoreType.DMA((K,)),
            pltpu.SemaphoreType.DMA((K,)),
            pltpu.SemaphoreType.DMA((K,)),
            pltpu.SemaphoreType.DMA((K,)),
            pltpu.VMEM((2, C, N), jnp.float32),
            pltpu.VMEM((2, C, N), jnp.float32),
            pltpu.VMEM((2, C, N), jnp.float32),
            pltpu.SemaphoreType.DMA((2,)),
            pltpu.SemaphoreType.DMA((2,)),
            pltpu.SemaphoreType.DMA((2,)),
        ],
        compiler_params=pltpu.CompilerParams(collective_id=0),
    )(x)
    return out


# device time: 241858 ns/iter; 1.7607x vs baseline; 1.7607x over previous
import os as _os

import jax
import jax.numpy as jnp
from jax import lax
from jax.experimental import pallas as pl
from jax.experimental.pallas import tpu as pltpu

MESH = pl.DeviceIdType.MESH

K = int(_os.environ.get("AR_K", "32"))
LAG = 2
D = 4


def kernel(x):
    M, N = x.shape
    HALF = M // 2
    C = HALF // K

    def body(x_hbm, out_hbm, comm_hbm, ssem1, rsem1, ssem2, rsem2,
             vx, vc, vo, lx, lc, st, vxs, vsend, lstg):
        my_x = lax.axis_index("x")
        my_y = lax.axis_index("y")
        y_nbr = (my_x, 1 - my_y)
        x_nbr = (1 - my_x, my_y)

        barrier = pltpu.get_barrier_semaphore()
        for nbr in (y_nbr, x_nbr):
            pl.semaphore_signal(barrier, inc=1, device_id=nbr,
                                device_id_type=MESH)
        pl.semaphore_wait(barrier, 2)

        def rows_mine(k):
            return pl.ds(my_x * HALF + k * C, C)

        def rows_other(k):
            return pl.ds((1 - my_x) * HALF + k * C, C)

        rdma1 = []

        def send1(k):
            s = k % D
            if k >= D:
                rdma1[k - D].wait_send()
            ld = pltpu.make_async_copy(x_hbm.at[rows_mine(k), :], vxs,
                                       lstg)
            ld.start()
            ld.wait()
            vsend[s] = vxs[...].astype(jnp.bfloat16)
            r = pltpu.make_async_remote_copy(
                src_ref=vsend.at[s],
                dst_ref=comm_hbm.at[rows_mine(k), :],
                send_sem=ssem1.at[k],
                recv_sem=rsem1.at[k],
                device_id=y_nbr,
                device_id_type=MESH,
            )
            r.start()
            rdma1.append(r)

        for k in range(min(D, K)):
            send1(k)

        rdma2 = []

        def forward(k):
            r = pltpu.make_async_remote_copy(
                src_ref=comm_hbm.at[rows_mine(k), :],
                dst_ref=comm_hbm.at[rows_mine(k), :],
                send_sem=ssem2.at[k],
                recv_sem=rsem2.at[k],
                device_id=x_nbr,
                device_id_type=MESH,
            )
            r.start()
            rdma2.append(r)

        tasks = []
        for k in range(K):
            tasks.append(("p1", k))
            if k >= LAG:
                tasks.append(("p2", k - LAG))
        for k in range(K - LAG, K):
            tasks.append(("p2", k))

        def task_rows(t):
            phase, k = tasks[t]
            return rows_mine(k) if phase == "p1" else rows_other(k)

        def start_xload(t):
            s = t % 2
            pltpu.make_async_copy(x_hbm.at[task_rows(t), :], vx.at[s],
                                  lx.at[s]).start()

        start_xload(0)
        for t in range(len(tasks)):
            s = t % 2
            phase, k = tasks[t]
            if phase == "p1":
                if k + D < K:
                    send1(k + D)
                rdma1[k].wait_recv()
                forward(k)
            else:
                rdma2[k].wait_recv()
            ld_c = pltpu.make_async_copy(comm_hbm.at[task_rows(t), :],
                                         vc.at[s], lc.at[s])
            ld_c.start()
            if t + 1 < len(tasks):
                start_xload(t + 1)
            pltpu.make_async_copy(x_hbm.at[task_rows(t), :], vx.at[s],
                                  lx.at[s]).wait()
            ld_c.wait()
            if t >= 2:
                pltpu.make_async_copy(vo.at[s], out_hbm.at[task_rows(t - 2), :],
                                      st.at[s]).wait()
            vo[s] = vx[s] + vc[s].astype(jnp.float32)
            pltpu.make_async_copy(vo.at[s], out_hbm.at[task_rows(t), :],
                                  st.at[s]).start()

        n_t = len(tasks)
        for t in (n_t - 2, n_t - 1):
            s = t % 2
            pltpu.make_async_copy(vo.at[s], out_hbm.at[task_rows(t), :],
                                  st.at[s]).wait()
        for k in range(max(K - D, 0), K):
            rdma1[k].wait_send()
        for k in range(K):
            rdma2[k].wait_send()

    out, _comm = pl.pallas_call(
        body,
        out_shape=[
            jax.ShapeDtypeStruct((M, N), jnp.float32),
            jax.ShapeDtypeStruct((M, N), jnp.bfloat16),
        ],
        in_specs=[pl.BlockSpec(memory_space=pltpu.HBM)],
        out_specs=[
            pl.BlockSpec(memory_space=pltpu.HBM),
            pl.BlockSpec(memory_space=pltpu.HBM),
        ],
        scratch_shapes=[
            pltpu.SemaphoreType.DMA((K,)),
            pltpu.SemaphoreType.DMA((K,)),
            pltpu.SemaphoreType.DMA((K,)),
            pltpu.SemaphoreType.DMA((K,)),
            pltpu.VMEM((2, C, N), jnp.float32),
            pltpu.VMEM((2, C, N), jnp.bfloat16),
            pltpu.VMEM((2, C, N), jnp.float32),
            pltpu.SemaphoreType.DMA((2,)),
            pltpu.SemaphoreType.DMA((2,)),
            pltpu.SemaphoreType.DMA((2,)),
            pltpu.VMEM((C, N), jnp.float32),
            pltpu.VMEM((D, C, N), jnp.bfloat16),
            pltpu.SemaphoreType.DMA,
        ],
        compiler_params=pltpu.CompilerParams(collective_id=0),
    )(x)
    return out
